# baseline (speedup 1.0000x reference)
import jax
import jax.numpy as jnp
from jax.experimental import pallas as pl
from jax.experimental.pallas import tpu as pltpu

MARGIN = 1.0
BLK = 128


def _loss_body(x_ref, lc_ref, lr_ref, out_ref):
    x = x_ref[...]
    n_total, dim = x.shape
    nb = n_total // BLK
    nrow = jax.lax.dot_general(
        jnp.ones((1, dim), jnp.float32), x * x,
        (((1,), (1,)), ((), ())),
        preferred_element_type=jnp.float32)

    acc_diag = jnp.zeros((BLK, BLK), jnp.float32)
    acc_off = jnp.zeros((BLK, BLK), jnp.float32)
    for i in range(nb):
        xi = x_ref[pl.ds(i * BLK, BLK), :]
        ni = jnp.transpose(nrow[:, i * BLK:(i + 1) * BLK], (1, 0))
        li = lc_ref[pl.ds(i * BLK, BLK), :]
        for j in range(i, nb):
            xj = x_ref[pl.ds(j * BLK, BLK), :]
            g = jax.lax.dot_general(
                xi, xj, (((1,), (1,)), ((), ())),
                preferred_element_type=jnp.float32)
            nj = nrow[:, j * BLK:(j + 1) * BLK]
            d = jnp.maximum(ni + nj - 2.0 * g, 1e-37)
            eq = li == lr_ref[:, pl.ds(j * BLK, BLK)]
            sqrt_d = d * jax.lax.rsqrt(d)
            neg = jnp.maximum(MARGIN - sqrt_d, 0.0)
            loss = jnp.where(eq, d, neg * neg)
            if i == j:
                acc_diag = acc_diag + loss
            else:
                acc_off = acc_off + loss
    total = jnp.sum(acc_diag + 2.0 * acc_off)
    scale = 1.0 / (n_total * (n_total - 1.0))
    out_ref[0] = total * scale


def kernel(embeddings_t, target_t):
    n, _ = embeddings_t.shape
    lc = target_t.reshape(n, 1)
    lr = target_t.reshape(1, n)
    out = pl.pallas_call(
        _loss_body,
        out_specs=pl.BlockSpec(memory_space=pltpu.SMEM),
        out_shape=jax.ShapeDtypeStruct((1,), jnp.float32),
    )(embeddings_t, lc, lr)
    return out[0]


# speedup vs baseline: 1.0217x; 1.0217x over previous
import jax
import jax.numpy as jnp
from jax.experimental import pallas as pl
from jax.experimental.pallas import tpu as pltpu

MARGIN = 1.0
BLK = 256


def _loss_body(x_ref, lc_ref, lr_ref, out_ref):
    x = x_ref[...]
    n_total, dim = x.shape
    nb = n_total // BLK
    nrow = jax.lax.dot_general(
        jnp.ones((1, dim), jnp.float32), x * x,
        (((1,), (1,)), ((), ())),
        preferred_element_type=jnp.float32)

    acc_diag = jnp.zeros((BLK, BLK), jnp.float32)
    acc_off = jnp.zeros((BLK, BLK), jnp.float32)
    for i in range(nb):
        xi = x_ref[pl.ds(i * BLK, BLK), :]
        ni = jnp.transpose(nrow[:, i * BLK:(i + 1) * BLK], (1, 0))
        li = lc_ref[pl.ds(i * BLK, BLK), :]
        for j in range(i, nb):
            xj = x_ref[pl.ds(j * BLK, BLK), :]
            g = jax.lax.dot_general(
                xi, xj, (((1,), (1,)), ((), ())),
                preferred_element_type=jnp.float32)
            nj = nrow[:, j * BLK:(j + 1) * BLK]
            d = jnp.maximum(ni + nj - 2.0 * g, 1e-37)
            eq = li == lr_ref[:, pl.ds(j * BLK, BLK)]
            sqrt_d = d * jax.lax.rsqrt(d)
            neg = jnp.maximum(MARGIN - sqrt_d, 0.0)
            loss = jnp.where(eq, d, neg * neg)
            if i == j:
                acc_diag = acc_diag + loss
            else:
                acc_off = acc_off + loss
    total = jnp.sum(acc_diag + 2.0 * acc_off)
    scale = 1.0 / (n_total * (n_total - 1.0))
    out_ref[0] = total * scale


def kernel(embeddings_t, target_t):
    n, _ = embeddings_t.shape
    lc = target_t.reshape(n, 1)
    lr = target_t.reshape(1, n)
    out = pl.pallas_call(
        _loss_body,
        out_specs=pl.BlockSpec(memory_space=pltpu.SMEM),
        out_shape=jax.ShapeDtypeStruct((1,), jnp.float32),
    )(embeddings_t, lc, lr)
    return out[0]
